# baseline (speedup 1.0000x reference)
import functools

import jax
import jax.numpy as jnp
from jax import lax
from jax.experimental import pallas as pl
from jax.experimental.pallas import tpu as pltpu
from jax.experimental.pallas import tpu_sc as plsc

NUM_INPUTS = 20
BATCH = 16384
LANES = 16
NUM_WORKERS = 32
B_PER_W = BATCH // NUM_WORKERS
GCHUNK = 128
NCHUNK = B_PER_W // GCHUNK
GROUPS_PER_CHUNK = GCHUNK // LANES

PACK_GRID = 4
PACK_BLOCK = BATCH // PACK_GRID

_LOGIT_LO = -16.118095
_LOGIT_HI = 15.942385


def _pack_body(x_ref, idx_ref):
  k = lax.broadcasted_iota(jnp.int32, (PACK_BLOCK, NUM_INPUTS), 1)
  bits = x_ref[...].astype(jnp.int32) << (NUM_INPUTS - 1 - k)
  idx_ref[...] = jnp.sum(bits, axis=1)


_pack_indices = pl.pallas_call(
    _pack_body,
    grid=(PACK_GRID,),
    in_specs=[pl.BlockSpec((PACK_BLOCK, NUM_INPUTS), lambda i: (i, 0))],
    out_specs=pl.BlockSpec((PACK_BLOCK,), lambda i: (i,)),
    out_shape=jax.ShapeDtypeStruct((BATCH,), jnp.int32),
)


def _gather_body(idx_hbm, lut_hbm, out_hbm, probs_hbm, logits_hbm,
                 idx_v, vals_v, out_v, probs_v, logits_v, semi, semg, semo):
  wid = lax.axis_index("s") * 2 + lax.axis_index("c")
  base = wid * B_PER_W

  pltpu.async_copy(idx_hbm.at[pl.ds(base, B_PER_W)], idx_v, semi).wait()

  gcopies = []
  for j in range(NCHUNK):
    gcopies.append(pltpu.async_copy(
        lut_hbm.at[idx_v.at[pl.ds(j * GCHUNK, GCHUNK)]],
        vals_v.at[pl.ds(j * GCHUNK, GCHUNK)], semg.at[j]))

  ocopies = []
  for j in range(NCHUNK):
    gcopies[j].wait()
    for g in range(GROUPS_PER_CHUNK):
      off = j * GCHUNK + g * LANES
      gval = vals_v[pl.ds(off, LANES)]
      p = 1.0 / (1.0 + jnp.exp(-gval))
      out_v[pl.ds(off, LANES)] = jnp.where(
          p >= 0.5, jnp.float32(1.0), jnp.float32(0.0))
      probs_v[pl.ds(off, LANES)] = p
      logits_v[pl.ds(off, LANES)] = 5.0 * jnp.clip(gval, _LOGIT_LO, _LOGIT_HI)
    src = pl.ds(j * GCHUNK, GCHUNK)
    dst = pl.ds(base + j * GCHUNK, GCHUNK)
    ocopies.append(pltpu.async_copy(out_v.at[src], out_hbm.at[dst],
                                    semo.at[3 * j]))
    ocopies.append(pltpu.async_copy(probs_v.at[src], probs_hbm.at[dst],
                                    semo.at[3 * j + 1]))
    ocopies.append(pltpu.async_copy(logits_v.at[src], logits_hbm.at[dst],
                                    semo.at[3 * j + 2]))
  for c in ocopies:
    c.wait()


_OUT = jax.ShapeDtypeStruct((BATCH,), jnp.float32)

_gather_sc = functools.partial(
    pl.kernel,
    out_type=(_OUT, _OUT, _OUT),
    mesh=plsc.VectorSubcoreMesh(core_axis_name="c", subcore_axis_name="s"),
    compiler_params=pltpu.CompilerParams(needs_layout_passes=False),
    scratch_types=[
        pltpu.VMEM((B_PER_W,), jnp.int32),
        pltpu.VMEM((B_PER_W,), jnp.float32),
        pltpu.VMEM((B_PER_W,), jnp.float32),
        pltpu.VMEM((B_PER_W,), jnp.float32),
        pltpu.VMEM((B_PER_W,), jnp.float32),
        pltpu.SemaphoreType.DMA,
        pltpu.SemaphoreType.DMA((NCHUNK,)),
        pltpu.SemaphoreType.DMA((3 * NCHUNK,)),
    ],
)(_gather_body)


@jax.jit
def kernel(x, lut_params):
  idx = _pack_indices(x)
  return _gather_sc(idx, lut_params)


# speedup vs baseline: 1.5042x; 1.5042x over previous
import functools

import jax
import jax.numpy as jnp
from jax import lax
from jax.experimental import pallas as pl
from jax.experimental.pallas import tpu as pltpu
from jax.experimental.pallas import tpu_sc as plsc

NUM_INPUTS = 20
BATCH = 16384
LANES = 16
NUM_WORKERS = 32
B_PER_W = BATCH // NUM_WORKERS
GCHUNK = 128
NCHUNK = B_PER_W // GCHUNK
GROUPS_PER_CHUNK = GCHUNK // LANES

_W = [float(2 ** (NUM_INPUTS - 1 - k)) for k in range(NUM_INPUTS)]

_LOGIT_LO = -16.118095
_LOGIT_HI = 15.942385


def _tree_sum(vals):
  while len(vals) > 1:
    nxt = [a + b for a, b in zip(vals[::2], vals[1::2])]
    if len(vals) % 2:
      nxt.append(vals[-1])
    vals = nxt
  return vals[0]


def _logic_unit_body(xt_hbm, lut_hbm, out_hbm, probs_hbm, logits_hbm,
                     x_v, idx_v, vals_v, out_v, probs_v, logits_v,
                     semx, semg, semo):
  wid = lax.axis_index("s") * 2 + lax.axis_index("c")
  base = wid * B_PER_W

  xcopies = []
  for j in range(NCHUNK):
    xcopies.append(pltpu.async_copy(
        xt_hbm.at[:, pl.ds(base + j * GCHUNK, GCHUNK)],
        x_v.at[:, pl.ds(j * GCHUNK, GCHUNK)], semx.at[j]))

  gcopies = []
  for j in range(NCHUNK):
    xcopies[j].wait()
    for g in range(GROUPS_PER_CHUNK):
      off = j * GCHUNK + g * LANES
      acc = _tree_sum([x_v[k, pl.ds(off, LANES)] * _W[k]
                       for k in range(NUM_INPUTS)])
      idx_v[pl.ds(off, LANES)] = acc.astype(jnp.int32)
    gcopies.append(pltpu.async_copy(
        lut_hbm.at[idx_v.at[pl.ds(j * GCHUNK, GCHUNK)]],
        vals_v.at[pl.ds(j * GCHUNK, GCHUNK)], semg.at[j]))

  ocopies = []
  for j in range(NCHUNK):
    gcopies[j].wait()
    for g in range(GROUPS_PER_CHUNK):
      off = j * GCHUNK + g * LANES
      gval = vals_v[pl.ds(off, LANES)]
      p = 1.0 / (1.0 + jnp.exp(-gval))
      out_v[pl.ds(off, LANES)] = jnp.where(
          p >= 0.5, jnp.float32(1.0), jnp.float32(0.0))
      probs_v[pl.ds(off, LANES)] = p
      logits_v[pl.ds(off, LANES)] = 5.0 * jnp.clip(gval, _LOGIT_LO, _LOGIT_HI)
    src = pl.ds(j * GCHUNK, GCHUNK)
    dst = pl.ds(base + j * GCHUNK, GCHUNK)
    ocopies.append(pltpu.async_copy(out_v.at[src], out_hbm.at[dst],
                                    semo.at[3 * j]))
    ocopies.append(pltpu.async_copy(probs_v.at[src], probs_hbm.at[dst],
                                    semo.at[3 * j + 1]))
    ocopies.append(pltpu.async_copy(logits_v.at[src], logits_hbm.at[dst],
                                    semo.at[3 * j + 2]))
  for c in ocopies:
    c.wait()


_OUT = jax.ShapeDtypeStruct((BATCH,), jnp.float32)

_logic_unit_sc = functools.partial(
    pl.kernel,
    out_type=(_OUT, _OUT, _OUT),
    mesh=plsc.VectorSubcoreMesh(core_axis_name="c", subcore_axis_name="s"),
    compiler_params=pltpu.CompilerParams(needs_layout_passes=False),
    scratch_types=[
        pltpu.VMEM((NUM_INPUTS, B_PER_W), jnp.float32),
        pltpu.VMEM((B_PER_W,), jnp.int32),
        pltpu.VMEM((B_PER_W,), jnp.float32),
        pltpu.VMEM((B_PER_W,), jnp.float32),
        pltpu.VMEM((B_PER_W,), jnp.float32),
        pltpu.VMEM((B_PER_W,), jnp.float32),
        pltpu.SemaphoreType.DMA((NCHUNK,)),
        pltpu.SemaphoreType.DMA((NCHUNK,)),
        pltpu.SemaphoreType.DMA((3 * NCHUNK,)),
    ],
)(_logic_unit_body)


@jax.jit
def kernel(x, lut_params):
  return _logic_unit_sc(x.T, lut_params)


# speedup vs baseline: 1.5224x; 1.0121x over previous
import functools

import jax
import jax.numpy as jnp
from jax import lax
from jax.experimental import pallas as pl
from jax.experimental.pallas import tpu as pltpu
from jax.experimental.pallas import tpu_sc as plsc

NUM_INPUTS = 20
BATCH = 16384
LANES = 16
NUM_WORKERS = 32
B_PER_W = BATCH // NUM_WORKERS
GCHUNK = 128
NCHUNK = B_PER_W // GCHUNK
GROUPS_PER_CHUNK = GCHUNK // LANES

_W = [float(2 ** (NUM_INPUTS - 1 - k)) for k in range(NUM_INPUTS)]

_LOGIT_LO = -16.118095
_LOGIT_HI = 15.942385


def _tree_sum(vals):
  while len(vals) > 1:
    nxt = [a + b for a, b in zip(vals[::2], vals[1::2])]
    if len(vals) % 2:
      nxt.append(vals[-1])
    vals = nxt
  return vals[0]


def _logic_unit_body(xt_hbm, lut_hbm, out_hbm, probs_hbm, logits_hbm,
                     x_v, idx_v, vals_v, out_v, probs_v, logits_v,
                     semx, semg, semo):
  wid = lax.axis_index("s") * 2 + lax.axis_index("c")
  base = wid * B_PER_W

  xcopies = []
  for j in range(NCHUNK):
    xcopies.append(pltpu.async_copy(
        xt_hbm.at[:, pl.ds(base + j * GCHUNK, GCHUNK)],
        x_v.at[:, pl.ds(j * GCHUNK, GCHUNK)], semx.at[j]))

  gcopies = []
  for j in range(NCHUNK):
    xcopies[j].wait()

    def pack_group(g, carry, j=j):
      off = pl.multiple_of(j * GCHUNK + g * LANES, LANES)
      acc = _tree_sum([x_v[k, pl.ds(off, LANES)] * _W[k]
                       for k in range(NUM_INPUTS)])
      idx_v[pl.ds(off, LANES)] = acc.astype(jnp.int32)
      return carry

    lax.fori_loop(0, GROUPS_PER_CHUNK, pack_group, 0, unroll=2)
    gcopies.append(pltpu.async_copy(
        lut_hbm.at[idx_v.at[pl.ds(j * GCHUNK, GCHUNK)]],
        vals_v.at[pl.ds(j * GCHUNK, GCHUNK)], semg.at[j]))

  ocopies = []
  for j in range(NCHUNK):
    gcopies[j].wait()

    def tail_group(g, carry, j=j):
      off = pl.multiple_of(j * GCHUNK + g * LANES, LANES)
      gval = vals_v[pl.ds(off, LANES)]
      p = 1.0 / (1.0 + jnp.exp(-gval))
      out_v[pl.ds(off, LANES)] = jnp.where(
          p >= 0.5, jnp.float32(1.0), jnp.float32(0.0))
      probs_v[pl.ds(off, LANES)] = p
      logits_v[pl.ds(off, LANES)] = 5.0 * jnp.clip(gval, _LOGIT_LO, _LOGIT_HI)
      return carry

    lax.fori_loop(0, GROUPS_PER_CHUNK, tail_group, 0, unroll=2)
    src = pl.ds(j * GCHUNK, GCHUNK)
    dst = pl.ds(base + j * GCHUNK, GCHUNK)
    ocopies.append(pltpu.async_copy(out_v.at[src], out_hbm.at[dst],
                                    semo.at[3 * j]))
    ocopies.append(pltpu.async_copy(probs_v.at[src], probs_hbm.at[dst],
                                    semo.at[3 * j + 1]))
    ocopies.append(pltpu.async_copy(logits_v.at[src], logits_hbm.at[dst],
                                    semo.at[3 * j + 2]))
  for c in ocopies:
    c.wait()


_OUT = jax.ShapeDtypeStruct((BATCH,), jnp.float32)

_logic_unit_sc = functools.partial(
    pl.kernel,
    out_type=(_OUT, _OUT, _OUT),
    mesh=plsc.VectorSubcoreMesh(core_axis_name="c", subcore_axis_name="s"),
    compiler_params=pltpu.CompilerParams(needs_layout_passes=False),
    scratch_types=[
        pltpu.VMEM((NUM_INPUTS, B_PER_W), jnp.float32),
        pltpu.VMEM((B_PER_W,), jnp.int32),
        pltpu.VMEM((B_PER_W,), jnp.float32),
        pltpu.VMEM((B_PER_W,), jnp.float32),
        pltpu.VMEM((B_PER_W,), jnp.float32),
        pltpu.VMEM((B_PER_W,), jnp.float32),
        pltpu.SemaphoreType.DMA((NCHUNK,)),
        pltpu.SemaphoreType.DMA((NCHUNK,)),
        pltpu.SemaphoreType.DMA((3 * NCHUNK,)),
    ],
)(_logic_unit_body)


@jax.jit
def kernel(x, lut_params):
  return _logic_unit_sc(x.T, lut_params)


# speedup vs baseline: 1.5242x; 1.0012x over previous
import functools

import jax
import jax.numpy as jnp
from jax import lax
from jax.experimental import pallas as pl
from jax.experimental.pallas import tpu as pltpu
from jax.experimental.pallas import tpu_sc as plsc

NUM_INPUTS = 20
BATCH = 16384
LANES = 16
NUM_WORKERS = 32
B_PER_W = BATCH // NUM_WORKERS
GCHUNK = 128
NCHUNK = B_PER_W // GCHUNK
GROUPS_PER_CHUNK = GCHUNK // LANES

_W = [float(2 ** (NUM_INPUTS - 1 - k)) for k in range(NUM_INPUTS)]

_LOGIT_LO = -16.118095
_LOGIT_HI = 15.942385


def _tree_sum(vals):
  while len(vals) > 1:
    nxt = [a + b for a, b in zip(vals[::2], vals[1::2])]
    if len(vals) % 2:
      nxt.append(vals[-1])
    vals = nxt
  return vals[0]


def _logic_unit_body(xt_hbm, lut_hbm, out_hbm, probs_hbm, logits_hbm,
                     x_v, idx_v, vals_v, out_v, probs_v, logits_v,
                     semx, semg, semo):
  wid = lax.axis_index("s") * 2 + lax.axis_index("c")
  base = wid * B_PER_W

  xcopies = []
  for j in range(NCHUNK):
    xcopies.append(pltpu.async_copy(
        xt_hbm.at[:, pl.ds(base + j * GCHUNK, GCHUNK)],
        x_v.at[:, pl.ds(j * GCHUNK, GCHUNK)], semx.at[j]))

  gcopies = []
  for j in range(NCHUNK):
    xcopies[j].wait()

    def pack_group(g, carry, j=j):
      off = pl.multiple_of(j * GCHUNK + g * LANES, LANES)
      acc = _tree_sum([x_v[k, pl.ds(off, LANES)] * _W[k]
                       for k in range(NUM_INPUTS)])
      idx_v[pl.ds(off, LANES)] = acc.astype(jnp.int32)
      return carry

    lax.fori_loop(0, GROUPS_PER_CHUNK, pack_group, 0, unroll=2)
    gcopies.append(pltpu.async_copy(
        lut_hbm.at[idx_v.at[pl.ds(j * GCHUNK, GCHUNK)]],
        vals_v.at[pl.ds(j * GCHUNK, GCHUNK)], semg.at[j]))

  ocopies = []
  for j in range(NCHUNK):
    gcopies[j].wait()

    def tail_group(g, carry, j=j):
      off = pl.multiple_of(j * GCHUNK + g * LANES, LANES)
      gval = vals_v[pl.ds(off, LANES)]
      p = 1.0 / (1.0 + jnp.exp(-gval))
      out_v[pl.ds(off, LANES)] = jnp.where(
          p >= 0.5, jnp.float32(1.0), jnp.float32(0.0))
      probs_v[pl.ds(off, LANES)] = p
      logits_v[pl.ds(off, LANES)] = 5.0 * jnp.clip(gval, _LOGIT_LO, _LOGIT_HI)
      return carry

    lax.fori_loop(0, GROUPS_PER_CHUNK, tail_group, 0, unroll=2)
    src = pl.ds(j * GCHUNK, GCHUNK)
    dst = pl.ds(base + j * GCHUNK, GCHUNK)
    ocopies.append(pltpu.async_copy(out_v.at[src], out_hbm.at[dst],
                                    semo.at[3 * j]))
    ocopies.append(pltpu.async_copy(probs_v.at[src], probs_hbm.at[dst],
                                    semo.at[3 * j + 1]))
    ocopies.append(pltpu.async_copy(logits_v.at[src], logits_hbm.at[dst],
                                    semo.at[3 * j + 2]))
  for c in ocopies:
    c.wait()


_OUT = jax.ShapeDtypeStruct((BATCH,), jnp.float32)

_logic_unit_sc = functools.partial(
    pl.kernel,
    out_type=(_OUT, _OUT, _OUT),
    mesh=plsc.VectorSubcoreMesh(core_axis_name="c", subcore_axis_name="s"),
    compiler_params=pltpu.CompilerParams(
        needs_layout_passes=False,
        disable_bounds_checks=True,
        disable_semaphore_checks=True,
    ),
    scratch_types=[
        pltpu.VMEM((NUM_INPUTS, B_PER_W), jnp.float32),
        pltpu.VMEM((B_PER_W,), jnp.int32),
        pltpu.VMEM((B_PER_W,), jnp.float32),
        pltpu.VMEM((B_PER_W,), jnp.float32),
        pltpu.VMEM((B_PER_W,), jnp.float32),
        pltpu.VMEM((B_PER_W,), jnp.float32),
        pltpu.SemaphoreType.DMA((NCHUNK,)),
        pltpu.SemaphoreType.DMA((NCHUNK,)),
        pltpu.SemaphoreType.DMA((3 * NCHUNK,)),
    ],
)(_logic_unit_body)


@jax.jit
def kernel(x, lut_params):
  return _logic_unit_sc(x.T, lut_params)


# speedup vs baseline: 1.6564x; 1.0867x over previous
import functools

import jax
import jax.numpy as jnp
from jax import lax
from jax.experimental import pallas as pl
from jax.experimental.pallas import tpu as pltpu
from jax.experimental.pallas import tpu_sc as plsc

NUM_INPUTS = 20
BATCH = 16384
LANES = 16
NUM_WORKERS = 32
B_PER_W = BATCH // NUM_WORKERS
GCHUNK = 128
NCHUNK = B_PER_W // GCHUNK
GROUPS_PER_CHUNK = GCHUNK // LANES

PACK_GRID = 4
PACK_BLOCK = BATCH // PACK_GRID

_LOGIT_LO = -16.118095
_LOGIT_HI = 15.942385


def _pack_body(xt_ref, idx_ref):
  k = lax.broadcasted_iota(jnp.int32, (NUM_INPUTS, PACK_BLOCK), 0)
  bits = xt_ref[...].astype(jnp.int32) << (NUM_INPUTS - 1 - k)
  idx_ref[...] = jnp.sum(bits, axis=0)


_pack_indices = pl.pallas_call(
    _pack_body,
    grid=(PACK_GRID,),
    in_specs=[pl.BlockSpec((NUM_INPUTS, PACK_BLOCK), lambda i: (0, i))],
    out_specs=pl.BlockSpec((PACK_BLOCK,), lambda i: (i,)),
    out_shape=jax.ShapeDtypeStruct((BATCH,), jnp.int32),
)


def _gather_body(idx_hbm, lut_hbm, out_hbm, probs_hbm, logits_hbm,
                 idx_v, vals_v, out_v, probs_v, logits_v, semi, semg, semo):
  wid = lax.axis_index("s") * 2 + lax.axis_index("c")
  base = wid * B_PER_W

  pltpu.async_copy(idx_hbm.at[pl.ds(base, B_PER_W)], idx_v, semi).wait()

  gcopies = []
  for j in range(NCHUNK):
    gcopies.append(pltpu.async_copy(
        lut_hbm.at[idx_v.at[pl.ds(j * GCHUNK, GCHUNK)]],
        vals_v.at[pl.ds(j * GCHUNK, GCHUNK)], semg.at[j]))

  ocopies = []
  for j in range(NCHUNK):
    gcopies[j].wait()

    def tail_group(g, carry, j=j):
      off = pl.multiple_of(j * GCHUNK + g * LANES, LANES)
      gval = vals_v[pl.ds(off, LANES)]
      p = 1.0 / (1.0 + jnp.exp(-gval))
      out_v[pl.ds(off, LANES)] = jnp.where(
          p >= 0.5, jnp.float32(1.0), jnp.float32(0.0))
      probs_v[pl.ds(off, LANES)] = p
      logits_v[pl.ds(off, LANES)] = 5.0 * jnp.clip(gval, _LOGIT_LO, _LOGIT_HI)
      return carry

    lax.fori_loop(0, GROUPS_PER_CHUNK, tail_group, 0, unroll=2)
    src = pl.ds(j * GCHUNK, GCHUNK)
    dst = pl.ds(base + j * GCHUNK, GCHUNK)
    ocopies.append(pltpu.async_copy(out_v.at[src], out_hbm.at[dst],
                                    semo.at[3 * j]))
    ocopies.append(pltpu.async_copy(probs_v.at[src], probs_hbm.at[dst],
                                    semo.at[3 * j + 1]))
    ocopies.append(pltpu.async_copy(logits_v.at[src], logits_hbm.at[dst],
                                    semo.at[3 * j + 2]))
  for c in ocopies:
    c.wait()


_OUT = jax.ShapeDtypeStruct((BATCH,), jnp.float32)

_gather_sc = functools.partial(
    pl.kernel,
    out_type=(_OUT, _OUT, _OUT),
    mesh=plsc.VectorSubcoreMesh(core_axis_name="c", subcore_axis_name="s"),
    compiler_params=pltpu.CompilerParams(needs_layout_passes=False),
    scratch_types=[
        pltpu.VMEM((B_PER_W,), jnp.int32),
        pltpu.VMEM((B_PER_W,), jnp.float32),
        pltpu.VMEM((B_PER_W,), jnp.float32),
        pltpu.VMEM((B_PER_W,), jnp.float32),
        pltpu.VMEM((B_PER_W,), jnp.float32),
        pltpu.SemaphoreType.DMA,
        pltpu.SemaphoreType.DMA((NCHUNK,)),
        pltpu.SemaphoreType.DMA((3 * NCHUNK,)),
    ],
)(_gather_body)


@jax.jit
def kernel(x, lut_params):
  idx = _pack_indices(x.T)
  return _gather_sc(idx, lut_params)
